# baseline (speedup 1.0000x reference)
import functools

import jax
import jax.numpy as jnp
from jax import lax
from jax.experimental import pallas as pl
from jax.experimental.pallas import tpu as pltpu
from jax.experimental.pallas import tpu_sc as plsc


def _tc_body(table_ref, out_ref):
    out_ref[...] = table_ref[...][None]


def _tc_broadcast(symbol_library, batch, seq_len, d_model, dtype):
    bs = 512
    grid = (seq_len // bs, batch)
    return pl.pallas_call(
        _tc_body,
        grid=grid,
        in_specs=[pl.BlockSpec((bs, d_model), lambda i, b: (i, 0))],
        out_specs=pl.BlockSpec((1, bs, d_model), lambda i, b: (b, i, 0)),
        out_shape=jax.ShapeDtypeStruct((batch, seq_len, d_model), dtype),
    )(symbol_library)


def _sc_broadcast(symbol_library, batch, seq_len, d_model, dtype):
    num_workers = 32
    rows_per_worker = seq_len // num_workers
    chunk = 32
    n_chunks = rows_per_worker // chunk
    nbuf = 2

    mesh = plsc.VectorSubcoreMesh(core_axis_name="c", subcore_axis_name="s")

    @functools.partial(
        pl.kernel,
        mesh=mesh,
        out_type=jax.ShapeDtypeStruct((batch, seq_len, d_model), dtype),
        scratch_types=[
            pltpu.VMEM((nbuf, chunk, d_model), jnp.float32),
            pltpu.SemaphoreType.DMA,
            pltpu.SemaphoreType.DMA,
        ],
    )
    def body(table_hbm, out_hbm, bufs, rsem, wsem):
        wid = lax.axis_index("s") * 2 + lax.axis_index("c")
        base = wid * rows_per_worker

        def start_read(c):
            return pltpu.async_copy(
                table_hbm.at[pl.ds(base + c * chunk, chunk)],
                bufs.at[c % nbuf], rsem)

        reads = {0: start_read(0)}
        writes = {}
        for c in range(n_chunks):
            reads[c].wait()
            if c + 1 < n_chunks:
                if c + 1 >= nbuf:
                    for w in writes.pop(c + 1 - nbuf):
                        w.wait()
                reads[c + 1] = start_read(c + 1)
            writes[c] = [
                pltpu.async_copy(
                    bufs.at[c % nbuf],
                    out_hbm.at[b, pl.ds(base + c * chunk, chunk)], wsem)
                for b in range(batch)
            ]
        for c in sorted(writes):
            for w in writes[c]:
                w.wait()

    return body(symbol_library)


def kernel(x, symbol_library):
    batch, seq_len, d_model = x.shape
    b_sc = batch // 2
    b_tc = batch - b_sc
    out_sc = _sc_broadcast(symbol_library, b_sc, seq_len, d_model, x.dtype)
    out_tc = _tc_broadcast(symbol_library, b_tc, seq_len, d_model, x.dtype)
    return jnp.concatenate([out_sc, out_tc], axis=0)


# speedup vs baseline: 1.9999x; 1.9999x over previous
import functools

import jax
import jax.numpy as jnp
from jax import lax
from jax.experimental import pallas as pl
from jax.experimental.pallas import tpu as pltpu
from jax.experimental.pallas import tpu_sc as plsc


def kernel(x, symbol_library):
    batch, seq_len, d_model = x.shape
    num_workers = 32
    rows_per_worker = seq_len // num_workers
    chunks = (48, 48, 16, 16)
    assert sum(chunks) == rows_per_worker
    starts = [sum(chunks[:i]) for i in range(len(chunks))]
    n_chunks = len(chunks)
    nbuf = 2
    bufrows = max(chunks)

    mesh = plsc.VectorSubcoreMesh(core_axis_name="c", subcore_axis_name="s")

    @functools.partial(
        pl.kernel,
        mesh=mesh,
        out_type=jax.ShapeDtypeStruct((batch, seq_len, d_model), x.dtype),
        scratch_types=[
            pltpu.VMEM((nbuf, bufrows, d_model), jnp.float32),
            pltpu.SemaphoreType.DMA,
            pltpu.SemaphoreType.DMA,
        ],
    )
    def body(table_hbm, out_hbm, bufs, rsem, wsem):
        wid = lax.axis_index("s") * 2 + lax.axis_index("c")
        base = wid * rows_per_worker

        def start_read(c):
            return pltpu.async_copy(
                table_hbm.at[pl.ds(base + starts[c], chunks[c])],
                bufs.at[c % nbuf, pl.ds(0, chunks[c])], rsem)

        reads = {0: start_read(0)}
        writes = {}
        for c in range(n_chunks):
            reads[c].wait()
            if c + 1 < n_chunks:
                if c + 1 >= nbuf:
                    for w in writes.pop(c + 1 - nbuf):
                        w.wait()
                reads[c + 1] = start_read(c + 1)
            writes[c] = [
                pltpu.async_copy(
                    bufs.at[c % nbuf, pl.ds(0, chunks[c])],
                    out_hbm.at[b, pl.ds(base + starts[c], chunks[c])], wsem)
                for b in range(batch)
            ]
        for c in sorted(writes):
            for w in writes[c]:
                w.wait()

    return body(symbol_library)
